# baseline (speedup 1.0000x reference)
import functools

import jax
import jax.numpy as jnp
from jax import lax
from jax.experimental import pallas as pl
from jax.experimental.pallas import tpu as pltpu
from jax.experimental.pallas import tpu_sc as plsc

EPS = 1e-5


def _norm_pack_body(tab_ref, gamma_ref, beta_ref, out_ref):
    x = tab_ref[...]
    mean = jnp.mean(x, axis=-1, keepdims=True)
    xc = x - mean
    var = jnp.mean(xc * xc, axis=-1, keepdims=True)
    nrm = xc * lax.rsqrt(var + EPS) * gamma_ref[...] + beta_ref[...]
    bits = lax.bitcast_convert_type(nrm, jnp.int32)
    half = bits.shape[-1] // 2
    lo = lax.shift_right_logical(bits[:, :half] + jnp.int32(0x8000), 16)
    hi = (bits[:, half:] + jnp.int32(0x8000)) & jnp.int32(-65536)
    out_ref[...] = hi | lo


def _normalize_pack(table, gamma, beta):
    V, D = table.shape
    R = 2000
    assert V % R == 0
    return pl.pallas_call(
        _norm_pack_body,
        grid=(V // R,),
        in_specs=[
            pl.BlockSpec((R, D), lambda i: (i, 0)),
            pl.BlockSpec((1, D), lambda i: (0, 0)),
            pl.BlockSpec((1, D), lambda i: (0, 0)),
        ],
        out_specs=pl.BlockSpec((R, D // 2), lambda i: (i, 0)),
        out_shape=jax.ShapeDtypeStruct((V, D // 2), jnp.int32),
    )(table, gamma.reshape(1, D), beta.reshape(1, D))


@functools.lru_cache(maxsize=None)
def _make_gather_unpack(V, D, N):
    info = plsc.get_sparse_core_info()
    NC, NS = info.num_cores, info.num_subcores
    NW = NC * NS
    assert N % NW == 0 and D % 32 == 0
    per_w = N // NW
    C = 128
    NBUF = 4
    W = D // 2
    assert per_w % (NBUF * C) == 0
    n_outer = per_w // (NBUF * C)
    mesh = plsc.VectorSubcoreMesh(core_axis_name="c", subcore_axis_name="s")

    @functools.partial(
        pl.kernel,
        mesh=mesh,
        compiler_params=pltpu.CompilerParams(use_tc_tiling_on_sc=False),
        out_type=jax.ShapeDtypeStruct((N * D,), jnp.int32),
        scratch_types=[
            pltpu.VMEM((per_w,), jnp.int32),
        ] + [pltpu.VMEM((C, W), jnp.int32)] * NBUF
          + [pltpu.VMEM((C * D,), jnp.int32)] * NBUF
          + [pltpu.SemaphoreType.DMA] * (2 * NBUF),
    )
    def gather_k(tab_hbm, idx_hbm, out_hbm, idx_v, *bufs_and_sems):
        pbs = bufs_and_sems[:NBUF]
        obs = bufs_and_sems[NBUF:2 * NBUF]
        sgs = bufs_and_sems[2 * NBUF:3 * NBUF]
        sws = bufs_and_sems[3 * NBUF:]
        wid = lax.axis_index("s") * NC + lax.axis_index("c")
        base = wid * per_w
        pltpu.sync_copy(idx_hbm.at[pl.ds(base, per_w)], idx_v)

        def issue_gather(i, b):
            pltpu.async_copy(
                tab_hbm.at[idx_v.at[pl.ds(i * C, C)]], pbs[b], sgs[b])

        def wait_gather(b):
            pltpu.make_async_copy(
                tab_hbm.at[idx_v.at[pl.ds(0, C)]], pbs[b], sgs[b]).wait()

        def issue_write(i, b):
            pltpu.async_copy(
                obs[b], out_hbm.at[pl.ds((base + i * C) * D, C * D)], sws[b])

        def wait_write(b):
            pltpu.make_async_copy(
                obs[b], out_hbm.at[pl.ds(0, C * D)], sws[b]).wait()

        def compute(b):
            pb = pbs[b]
            ob = obs[b]
            mask = jnp.int32(-65536)

            def row(r, carry):
                o = r * D
                for jw in range(W // 16):
                    w = pb[r, pl.ds(16 * jw, 16)]
                    ob[pl.ds(o + 16 * jw, 16)] = lax.shift_left(w, 16)
                    ob[pl.ds(o + W + 16 * jw, 16)] = w & mask
                return carry

            lax.fori_loop(0, C, row, 0)

        for b in range(NBUF):
            issue_gather(b, b)

        def body(t, carry):
            i0 = t * NBUF
            for b in range(NBUF):
                wait_gather(b)
                compute(b)
                issue_write(i0 + b, b)
            for b in range(NBUF):
                wait_write(b)
                issue_gather(i0 + NBUF + b, b)
            return carry

        lax.fori_loop(0, n_outer - 1, body, 0)
        i0 = (n_outer - 1) * NBUF
        for b in range(NBUF):
            wait_gather(b)
            compute(b)
            issue_write(i0 + b, b)
        for b in range(NBUF):
            wait_write(b)

    return gather_k


def kernel(x, table, gamma, beta):
    B, L = x.shape
    V, D = table.shape
    packed = _normalize_pack(table, gamma, beta)
    flat = x.reshape(-1).astype(jnp.int32)
    out = _make_gather_unpack(V, D, B * L)(packed, flat)
    return lax.bitcast_convert_type(out, jnp.float32).reshape(B, L, D)


# speedup vs baseline: 2.5346x; 2.5346x over previous
import functools

import jax
import jax.numpy as jnp
from jax import lax
from jax.experimental import pallas as pl
from jax.experimental.pallas import tpu as pltpu
from jax.experimental.pallas import tpu_sc as plsc

EPS = 1e-5


def _norm_body(tab_ref, gamma_ref, beta_ref, out_ref):
    xv = tab_ref[...]
    mean = jnp.mean(xv, axis=-1, keepdims=True)
    xc = xv - mean
    var = jnp.mean(xc * xc, axis=-1, keepdims=True)
    out_ref[...] = xc * lax.rsqrt(var + EPS) * gamma_ref[...] + beta_ref[...]


def _normalize_table(table, gamma, beta):
    V, D = table.shape
    R = 4000
    assert V % R == 0
    return pl.pallas_call(
        _norm_body,
        grid=(V // R,),
        in_specs=[
            pl.BlockSpec((R, D), lambda i: (i, 0)),
            pl.BlockSpec((1, D), lambda i: (0, 0)),
            pl.BlockSpec((1, D), lambda i: (0, 0)),
        ],
        out_specs=pl.BlockSpec((R, D), lambda i: (i, 0)),
        out_shape=jax.ShapeDtypeStruct((V, D), jnp.float32),
    )(table, gamma.reshape(1, D), beta.reshape(1, D))


@functools.lru_cache(maxsize=None)
def _make_gather(V, D, N):
    info = plsc.get_sparse_core_info()
    NC, NS = info.num_cores, info.num_subcores
    NW = NC * NS
    assert N % NW == 0
    per_w = N // NW
    C = 128
    NBUF = 5
    assert per_w % (NBUF * C) == 0
    n_outer = per_w // (NBUF * C)
    mesh = plsc.VectorSubcoreMesh(core_axis_name="c", subcore_axis_name="s")

    @functools.partial(
        pl.kernel,
        mesh=mesh,
        out_type=jax.ShapeDtypeStruct((N, D), jnp.float32),
        scratch_types=[
            pltpu.VMEM((per_w,), jnp.int32),
        ] + [pltpu.VMEM((C, D), jnp.float32)] * NBUF
          + [pltpu.SemaphoreType.DMA] * (2 * NBUF),
    )
    def gather_k(tab_hbm, idx_hbm, out_hbm, idx_v, *bufs_and_sems):
        rows = bufs_and_sems[:NBUF]
        sgs = bufs_and_sems[NBUF:2 * NBUF]
        sws = bufs_and_sems[2 * NBUF:]
        wid = lax.axis_index("s") * NC + lax.axis_index("c")
        base = wid * per_w
        pltpu.sync_copy(idx_hbm.at[pl.ds(base, per_w)], idx_v)

        def issue_gather(i, b):
            pltpu.async_copy(
                tab_hbm.at[idx_v.at[pl.ds(i * C, C)]], rows[b], sgs[b])

        def wait_gather(b):
            pltpu.make_async_copy(
                tab_hbm.at[idx_v.at[pl.ds(0, C)]], rows[b], sgs[b]).wait()

        def issue_write(i, b):
            pltpu.async_copy(rows[b], out_hbm.at[pl.ds(base + i * C, C)], sws[b])

        def wait_write(b):
            pltpu.make_async_copy(rows[b], out_hbm.at[pl.ds(0, C)], sws[b]).wait()

        for b in range(NBUF):
            issue_gather(b, b)

        def body(t, carry):
            i0 = t * NBUF
            for b in range(NBUF):
                wait_gather(b)
                issue_write(i0 + b, b)
            for b in range(NBUF):
                wait_write(b)
                issue_gather(i0 + NBUF + b, b)
            return carry

        lax.fori_loop(0, n_outer - 1, body, 0)
        i0 = (n_outer - 1) * NBUF
        for b in range(NBUF):
            wait_gather(b)
            issue_write(i0 + b, b)
        for b in range(NBUF):
            wait_write(b)

    return gather_k


def kernel(x, table, gamma, beta):
    B, L = x.shape
    V, D = table.shape
    norm = _normalize_table(table, gamma, beta)
    flat = x.reshape(-1).astype(jnp.int32)
    out = _make_gather(V, D, B * L)(norm, flat)
    return out.reshape(B, L, D)


# speedup vs baseline: 2.5486x; 1.0055x over previous
import functools

import jax
import jax.numpy as jnp
from jax import lax
from jax.experimental import pallas as pl
from jax.experimental.pallas import tpu as pltpu
from jax.experimental.pallas import tpu_sc as plsc

EPS = 1e-5


def _norm_body(tab_ref, gamma_ref, beta_ref, out_ref):
    xv = tab_ref[...]
    mean = jnp.mean(xv, axis=-1, keepdims=True)
    xc = xv - mean
    var = jnp.mean(xc * xc, axis=-1, keepdims=True)
    out_ref[...] = xc * lax.rsqrt(var + EPS) * gamma_ref[...] + beta_ref[...]


def _normalize_table(table, gamma, beta):
    V, D = table.shape
    R = 4000
    assert V % R == 0
    return pl.pallas_call(
        _norm_body,
        grid=(V // R,),
        in_specs=[
            pl.BlockSpec((R, D), lambda i: (i, 0)),
            pl.BlockSpec((1, D), lambda i: (0, 0)),
            pl.BlockSpec((1, D), lambda i: (0, 0)),
        ],
        out_specs=pl.BlockSpec((R, D), lambda i: (i, 0)),
        out_shape=jax.ShapeDtypeStruct((V, D), jnp.float32),
    )(table, gamma.reshape(1, D), beta.reshape(1, D))


@functools.lru_cache(maxsize=None)
def _make_gather(V, D, N):
    info = plsc.get_sparse_core_info()
    NC, NS = info.num_cores, info.num_subcores
    NW = NC * NS
    assert N % NW == 0
    per_w = N // NW
    C = 160
    NBUF = 5
    assert per_w % (NBUF * C) == 0
    n_outer = per_w // (NBUF * C)
    mesh = plsc.VectorSubcoreMesh(core_axis_name="c", subcore_axis_name="s")

    @functools.partial(
        pl.kernel,
        mesh=mesh,
        out_type=jax.ShapeDtypeStruct((N, D), jnp.float32),
        scratch_types=[
            pltpu.VMEM((per_w,), jnp.int32),
        ] + [pltpu.VMEM((C, D), jnp.float32)] * NBUF
          + [pltpu.SemaphoreType.DMA] * (2 * NBUF),
    )
    def gather_k(tab_hbm, idx_hbm, out_hbm, idx_v, *bufs_and_sems):
        rows = bufs_and_sems[:NBUF]
        sgs = bufs_and_sems[NBUF:2 * NBUF]
        sws = bufs_and_sems[2 * NBUF:]
        wid = lax.axis_index("s") * NC + lax.axis_index("c")
        base = wid * per_w
        pltpu.sync_copy(idx_hbm.at[pl.ds(base, per_w)], idx_v)

        def issue_gather(i, b):
            pltpu.async_copy(
                tab_hbm.at[idx_v.at[pl.ds(i * C, C)]], rows[b], sgs[b])

        def wait_gather(b):
            pltpu.make_async_copy(
                tab_hbm.at[idx_v.at[pl.ds(0, C)]], rows[b], sgs[b]).wait()

        def issue_write(i, b):
            pltpu.async_copy(rows[b], out_hbm.at[pl.ds(base + i * C, C)], sws[b])

        def wait_write(b):
            pltpu.make_async_copy(rows[b], out_hbm.at[pl.ds(0, C)], sws[b]).wait()

        for b in range(NBUF):
            issue_gather(b, b)

        def body(t, carry):
            i0 = t * NBUF
            for b in range(NBUF):
                wait_gather(b)
                issue_write(i0 + b, b)
            for b in range(NBUF):
                wait_write(b)
                issue_gather(i0 + NBUF + b, b)
            return carry

        lax.fori_loop(0, n_outer - 1, body, 0)
        i0 = (n_outer - 1) * NBUF
        for b in range(NBUF):
            wait_gather(b)
            issue_write(i0 + b, b)
        for b in range(NBUF):
            wait_write(b)

    return gather_k


def kernel(x, table, gamma, beta):
    B, L = x.shape
    V, D = table.shape
    norm = _normalize_table(table, gamma, beta)
    flat = x.reshape(-1).astype(jnp.int32)
    out = _make_gather(V, D, B * L)(norm, flat)
    return out.reshape(B, L, D)


# speedup vs baseline: 2.5675x; 1.0074x over previous
import functools

import jax
import jax.numpy as jnp
from jax import lax
from jax.experimental import pallas as pl
from jax.experimental.pallas import tpu as pltpu
from jax.experimental.pallas import tpu_sc as plsc

EPS = 1e-5


def _norm_body(tab_ref, gamma_ref, beta_ref, out_ref):
    xv = tab_ref[...]
    mean = jnp.mean(xv, axis=-1, keepdims=True)
    xc = xv - mean
    var = jnp.mean(xc * xc, axis=-1, keepdims=True)
    out_ref[...] = xc * lax.rsqrt(var + EPS) * gamma_ref[...] + beta_ref[...]


def _normalize_table(table, gamma, beta):
    V, D = table.shape
    R = 5000
    assert V % R == 0
    return pl.pallas_call(
        _norm_body,
        grid=(V // R,),
        in_specs=[
            pl.BlockSpec((R, D), lambda i: (i, 0)),
            pl.BlockSpec((1, D), lambda i: (0, 0)),
            pl.BlockSpec((1, D), lambda i: (0, 0)),
        ],
        out_specs=pl.BlockSpec((R, D), lambda i: (i, 0)),
        out_shape=jax.ShapeDtypeStruct((V, D), jnp.float32),
    )(table, gamma.reshape(1, D), beta.reshape(1, D))


@functools.lru_cache(maxsize=None)
def _make_gather(V, D, N):
    info = plsc.get_sparse_core_info()
    NC, NS = info.num_cores, info.num_subcores
    NW = NC * NS
    assert N % NW == 0
    per_w = N // NW
    C = 160
    NBUF = 5
    assert per_w % (NBUF * C) == 0
    n_outer = per_w // (NBUF * C)
    mesh = plsc.VectorSubcoreMesh(core_axis_name="c", subcore_axis_name="s")

    @functools.partial(
        pl.kernel,
        mesh=mesh,
        out_type=jax.ShapeDtypeStruct((N, D), jnp.float32),
        scratch_types=[
            pltpu.VMEM((per_w,), jnp.int32),
        ] + [pltpu.VMEM((C, D), jnp.float32)] * NBUF
          + [pltpu.SemaphoreType.DMA] * (2 * NBUF),
    )
    def gather_k(tab_hbm, idx_hbm, out_hbm, idx_v, *bufs_and_sems):
        rows = bufs_and_sems[:NBUF]
        sgs = bufs_and_sems[NBUF:2 * NBUF]
        sws = bufs_and_sems[2 * NBUF:]
        wid = lax.axis_index("s") * NC + lax.axis_index("c")
        base = wid * per_w
        pltpu.sync_copy(idx_hbm.at[pl.ds(base, per_w)], idx_v)

        def issue_gather(i, b):
            pltpu.async_copy(
                tab_hbm.at[idx_v.at[pl.ds(i * C, C)]], rows[b], sgs[b])

        def wait_gather(b):
            pltpu.make_async_copy(
                tab_hbm.at[idx_v.at[pl.ds(0, C)]], rows[b], sgs[b]).wait()

        def issue_write(i, b):
            pltpu.async_copy(rows[b], out_hbm.at[pl.ds(base + i * C, C)], sws[b])

        def wait_write(b):
            pltpu.make_async_copy(rows[b], out_hbm.at[pl.ds(0, C)], sws[b]).wait()

        for b in range(NBUF):
            issue_gather(b, b)

        def body(t, carry):
            i0 = t * NBUF
            for b in range(NBUF):
                wait_gather(b)
                issue_write(i0 + b, b)
            for b in range(NBUF):
                wait_write(b)
                issue_gather(i0 + NBUF + b, b)
            return carry

        lax.fori_loop(0, n_outer - 1, body, 0)
        i0 = (n_outer - 1) * NBUF
        for b in range(NBUF):
            wait_gather(b)
            issue_write(i0 + b, b)
        for b in range(NBUF):
            wait_write(b)

    return gather_k


def kernel(x, table, gamma, beta):
    B, L = x.shape
    V, D = table.shape
    norm = _normalize_table(table, gamma, beta)
    flat = x.reshape(-1).astype(jnp.int32)
    out = _make_gather(V, D, B * L)(norm, flat)
    return out.reshape(B, L, D)
